# Initial kernel scaffold; baseline (speedup 1.0000x reference)
import jax
import jax.numpy as jnp
from jax.experimental import pallas as pl


def kernel(x, edge_attr, u, params, edge_index, batch):
    raise NotImplementedError("write your pallas kernel here")


# speedup vs baseline: 3.1837x; 3.1837x over previous
import functools

import jax
import jax.numpy as jnp
from jax import lax
from jax.experimental import pallas as pl
from jax.experimental.pallas import tpu as pltpu
from jax.experimental.pallas import tpu_sc as plsc

N = 10000
E = 320000
G = 16
D = 128
H = 128

NC = 2
NS = 16
NW = NC * NS
EPW = E // NW
CG = 80
NCHUNK = EPW // CG
NGRP = CG // 16
BINBLK = 2000

_F32MIN = -3.4e38


def _bfr(a):
    ui = lax.bitcast_convert_type(a, jnp.int32)
    bias = 0x7FFF + ((ui >> 16) & 1)
    r = (ui + bias) & ~0xFFFF
    return lax.bitcast_convert_type(r, jnp.float32)


def _vgather(v, idx):
    return lax.gather(
        v, idx[:, None],
        lax.GatherDimensionNumbers(
            offset_dims=(), collapsed_slice_dims=(0,), start_index_map=(0,)),
        slice_sizes=(1,), mode=lax.GatherScatterMode.PROMISE_IN_BOUNDS)


def _edge_body(a_hbm, b_hbm, row_hbm, col_hbm, ea_hbm, wc_hbm,
               sum_hbm, max_hbm, cnt_hbm, se_hbm,
               sumbin, maxbin, cntbin, rbuf, cbuf, ebuf, sebuf,
               abuf, bbuf, wcbuf, sema, semb):
    wid = lax.axis_index("s") * NC + lax.axis_index("c")
    ebase = wid * EPW

    iota16 = lax.iota(jnp.int32, 16)
    zero16 = jnp.zeros((16,), jnp.float32)

    def init_body(i, _):
        sumbin[pl.ds(i * 16, 16)] = zero16
        cntbin[pl.ds(i * 16, 16)] = zero16
        maxbin[pl.ds(i * 16, 16)] = jnp.full((16,), _F32MIN, jnp.float32)
        return 0
    lax.fori_loop(0, N // 16, init_body, 0)

    pltpu.sync_copy(wc_hbm, wcbuf)
    w3 = [wcbuf[0, pl.ds(k * 16, 16)] for k in range(8)]
    w2 = [wcbuf[1, pl.ds(k * 16, 16)] for k in range(8)]
    be2v = wcbuf[2, pl.ds(0, 16)]

    def chunk_body(ci, _):
        off = ebase + ci * CG
        pltpu.sync_copy(row_hbm.at[pl.ds(off, CG)], rbuf)
        pltpu.sync_copy(col_hbm.at[pl.ds(off, CG)], cbuf)
        pltpu.sync_copy(ea_hbm.at[pl.ds(off, CG)], ebuf)
        cpa = pltpu.async_copy(a_hbm.at[rbuf], abuf, sema)
        cpb = pltpu.async_copy(b_hbm.at[cbuf], bbuf, semb)
        cpa.wait()
        cpb.wait()

        def group_body(g, _):
            cvec = cbuf[pl.ds(g * 16, 16)]
            evec = _bfr(ebuf[pl.ds(g * 16, 16)])
            sv = zero16
            for e in range(16):
                eidx = g * 16 + e
                eav = _vgather(evec, jnp.full((16,), e, jnp.int32))
                acc = zero16
                for k in range(8):
                    va = abuf[eidx, pl.ds(k * 16, 16)]
                    vb = bbuf[eidx, pl.ds(k * 16, 16)]
                    t = va + vb + eav * w3[k]
                    t = _bfr(jnp.maximum(t, 0.0))
                    acc = acc + t * w2[k]
                for d in (8, 4, 2, 1):
                    acc = acc + _vgather(acc, iota16 ^ d)
                sv = jnp.where(iota16 == e, acc, sv)
            sv = sv + be2v
            sebuf[pl.ds(g * 16, 16)] = sv

            sums = sv
            maxs = sv
            cnts = jnp.ones((16,), jnp.float32)
            bad = jnp.zeros((16,), jnp.bool_)
            for d in range(1, 16):
                idx = (iota16 + d) & 15
                same = cvec == _vgather(cvec, idx)
                svg = _vgather(sv, idx)
                wrap = iota16 >= (16 - d)
                bad = bad | (same & wrap)
                sums = sums + jnp.where(same, svg, 0.0)
                cnts = cnts + jnp.where(same, 1.0, 0.0)
                maxs = jnp.where(same, jnp.maximum(maxs, svg), maxs)
            first = ~bad
            plsc.addupdate_scatter(sumbin, [cvec], sums, mask=first)
            plsc.addupdate_scatter(cntbin, [cvec], cnts, mask=first)
            old = plsc.load_gather(maxbin, [cvec])
            plsc.store_scatter(maxbin, [cvec], jnp.maximum(old, maxs),
                               mask=first)
            return 0

        lax.fori_loop(0, NGRP, group_body, 0)
        pltpu.sync_copy(sebuf, se_hbm.at[pl.ds(off, CG)])
        return 0

    lax.fori_loop(0, NCHUNK, chunk_body, 0)

    pltpu.sync_copy(sumbin, sum_hbm.at[wid])
    pltpu.sync_copy(maxbin, max_hbm.at[wid])
    pltpu.sync_copy(cntbin, cnt_hbm.at[wid])


_edge_call = pl.kernel(
    _edge_body,
    out_type=(
        jax.ShapeDtypeStruct((NW, N), jnp.float32),
        jax.ShapeDtypeStruct((NW, N), jnp.float32),
        jax.ShapeDtypeStruct((NW, N), jnp.float32),
        jax.ShapeDtypeStruct((E,), jnp.float32),
    ),
    mesh=plsc.VectorSubcoreMesh(core_axis_name="c", subcore_axis_name="s"),
    compiler_params=pltpu.CompilerParams(needs_layout_passes=False),
    scratch_types=[
        pltpu.VMEM((N,), jnp.float32),
        pltpu.VMEM((N,), jnp.float32),
        pltpu.VMEM((N,), jnp.float32),
        pltpu.VMEM((CG,), jnp.int32),
        pltpu.VMEM((CG,), jnp.int32),
        pltpu.VMEM((CG,), jnp.float32),
        pltpu.VMEM((CG,), jnp.float32),
        pltpu.VMEM((CG, H), jnp.float32),
        pltpu.VMEM((CG, H), jnp.float32),
        pltpu.VMEM((3, H), jnp.float32),
        pltpu.SemaphoreType.DMA,
        pltpu.SemaphoreType.DMA,
    ],
)


BN = 2000
NB = N // BN
_HI = lax.Precision.HIGHEST


def _dot(a, b):
    return jnp.dot(a, b, preferred_element_type=jnp.float32, precision=_HI)


def _dot0(a, b):
    return lax.dot_general(a, b, (((0,), (0,)), ((), ())),
                           preferred_element_type=jnp.float32, precision=_HI)


def _full_spec(shape):
    return pl.BlockSpec(shape, lambda i: tuple(0 for _ in shape))


def _prep_body(x_ref, wa_ref, wb_ref, ba_ref, a_ref, b_ref):
    xv = _bfr(x_ref[...])
    a_ref[...] = _dot(xv, wa_ref[...]) + ba_ref[...]
    b_ref[...] = _dot(xv, wb_ref[...])


def _prep(x, wa, wb, ba):
    row_spec = pl.BlockSpec((BN, D), lambda i: (i, 0))
    return pl.pallas_call(
        _prep_body,
        grid=(NB,),
        in_specs=[row_spec, _full_spec((D, D)), _full_spec((D, D)),
                  _full_spec((1, H))],
        out_specs=(row_spec, row_spec),
        out_shape=(jax.ShapeDtypeStruct((N, D), jnp.float32),
                   jax.ShapeDtypeStruct((N, D), jnp.float32)),
    )(x, wa, wb, ba)


def _node_common(x_ref, ps_ref, pm_ref, pc_ref, batch_ref, u_ref,
                 wn1a_ref, wrow_ref, wu_ref, bn1_ref, wn2_ref, bn2_ref):
    s = jnp.sum(ps_ref[0], axis=0, keepdims=True)
    c = jnp.sum(pc_ref[0], axis=0, keepdims=True)
    m = jnp.max(pm_ref[0], axis=0, keepdims=True)
    m = jnp.where(c > 0.0, m, 0.0)
    mean = s / jnp.maximum(c, 1.0)

    wrow = wrow_ref[...]
    onehot = (batch_ref[...] ==
              lax.broadcasted_iota(jnp.int32, (BN, G), 1)).astype(jnp.float32)
    uw = _dot(_bfr(u_ref[...]), wu_ref[...])

    h = (_dot(_bfr(x_ref[...]), wn1a_ref[...])
         + _dot0(_bfr(s), wrow[0:1])
         + _dot0(_bfr(m), wrow[1:2])
         + _dot0(_bfr(mean), wrow[2:3])
         + _dot(onehot, uw)
         + bn1_ref[...])
    h = _bfr(jnp.maximum(h, 0.0))
    return _dot(h, wn2_ref[...]) + bn2_ref[...]


def _node_fused_body(x_ref, ps_ref, pm_ref, pc_ref, batch_ref, u_ref,
                     wn1a_ref, wrow_ref, wu_ref, bn1_ref, wn2_ref, bn2_ref,
                     wea_ref, web_ref, bea_ref,
                     xn_ref, a2_ref, b2_ref):
    xn = _node_common(x_ref, ps_ref, pm_ref, pc_ref, batch_ref, u_ref,
                      wn1a_ref, wrow_ref, wu_ref, bn1_ref, wn2_ref, bn2_ref)
    xn_ref[...] = xn
    xnr = _bfr(xn)
    a2_ref[...] = _dot(xnr, wea_ref[...]) + bea_ref[...]
    b2_ref[...] = _dot(xnr, web_ref[...])


def _node_plain_body(x_ref, ps_ref, pm_ref, pc_ref, batch_ref, u_ref,
                     wn1a_ref, wrow_ref, wu_ref, bn1_ref, wn2_ref, bn2_ref,
                     xn_ref):
    xn_ref[...] = _node_common(x_ref, ps_ref, pm_ref, pc_ref, batch_ref,
                               u_ref, wn1a_ref, wrow_ref, wu_ref, bn1_ref,
                               wn2_ref, bn2_ref)


def _node_specs():
    row_spec = pl.BlockSpec((BN, D), lambda i: (i, 0))
    part_spec = pl.BlockSpec((1, NW, BN), lambda i: (i, 0, 0))
    batch_spec = pl.BlockSpec((BN, 1), lambda i: (i, 0))
    wspecs = [_full_spec((G, 1)), _full_spec((D, D)), _full_spec((3, H)),
              _full_spec((1, H)), _full_spec((1, H)), _full_spec((D, D)),
              _full_spec((1, D))]
    return [row_spec, part_spec, part_spec, part_spec, batch_spec] + wspecs


def _node_fused(x, ps, pm, pc, batch2, u, wn1a, wrow, wu, bn1, wn2, bn2,
                wea, web, bea):
    row_spec = pl.BlockSpec((BN, D), lambda i: (i, 0))
    return pl.pallas_call(
        _node_fused_body,
        grid=(NB,),
        in_specs=_node_specs() + [_full_spec((D, D)), _full_spec((D, D)),
                                  _full_spec((1, H))],
        out_specs=(row_spec, row_spec, row_spec),
        out_shape=(jax.ShapeDtypeStruct((N, D), jnp.float32),
                   jax.ShapeDtypeStruct((N, D), jnp.float32),
                   jax.ShapeDtypeStruct((N, D), jnp.float32)),
    )(x, ps, pm, pc, batch2, u, wn1a, wrow, wu, bn1, wn2, bn2, wea, web, bea)


def _node_plain(x, ps, pm, pc, batch2, u, wn1a, wrow, wu, bn1, wn2, bn2):
    row_spec = pl.BlockSpec((BN, D), lambda i: (i, 0))
    return pl.pallas_call(
        _node_plain_body,
        grid=(NB,),
        in_specs=_node_specs(),
        out_specs=row_spec,
        out_shape=jax.ShapeDtypeStruct((N, D), jnp.float32),
    )(x, ps, pm, pc, batch2, u, wn1a, wrow, wu, bn1, wn2, bn2)


def _pool_body(x_ref, batch_ref, addp_ref, cnt_ref, maxp_ref):
    i = pl.program_id(0)
    xv = x_ref[...]
    bv = batch_ref[...]
    onehot = (bv == lax.broadcasted_iota(jnp.int32, (BN, G), 1)
              ).astype(jnp.float32)
    addp = _dot0(onehot, xv)
    cnt = _dot0(onehot, jnp.ones((BN, 1), jnp.float32))
    rows = []
    for g in range(G):
        rows.append(jnp.max(jnp.where(bv == g, xv, _F32MIN), axis=0,
                            keepdims=True))
    maxp = jnp.concatenate(rows, axis=0)

    @pl.when(i == 0)
    def _():
        addp_ref[...] = jnp.zeros((G, D), jnp.float32)
        cnt_ref[...] = jnp.zeros((G, 1), jnp.float32)
        maxp_ref[...] = jnp.full((G, D), _F32MIN, jnp.float32)

    addp_ref[...] += addp
    cnt_ref[...] += cnt
    maxp_ref[...] = jnp.maximum(maxp_ref[...], maxp)


def _pool(x, batch2):
    return pl.pallas_call(
        _pool_body,
        grid=(NB,),
        in_specs=[pl.BlockSpec((BN, D), lambda i: (i, 0)),
                  pl.BlockSpec((BN, 1), lambda i: (i, 0))],
        out_specs=(_full_spec((G, D)), _full_spec((G, 1)),
                   _full_spec((G, D))),
        out_shape=(jax.ShapeDtypeStruct((G, D), jnp.float32),
                   jax.ShapeDtypeStruct((G, 1), jnp.float32),
                   jax.ShapeDtypeStruct((G, D), jnp.float32)),
    )(x, batch2)


def _outmlp_body(addp_ref, cnt_ref, maxp_ref, u_ref,
                 w1a_ref, w1b_ref, w1c_ref, w1u_ref, b1_ref,
                 w2_ref, b2_ref, w3_ref, b3_ref, w4_ref, b4_ref, out_ref):
    addp = addp_ref[...]
    cnt = cnt_ref[...]
    maxp = jnp.where(cnt > 0.0, maxp_ref[...], 0.0)
    meanp = addp / jnp.maximum(cnt, 1.0)
    h = (_dot(_bfr(addp), w1a_ref[...]) + _dot(_bfr(meanp), w1b_ref[...])
         + _dot(_bfr(maxp), w1c_ref[...]) + _dot(_bfr(u_ref[...]),
                                                 w1u_ref[...])
         + b1_ref[...])
    h = _bfr(jnp.maximum(h, 0.0))
    h = _bfr(jnp.maximum(_dot(h, w2_ref[...]) + b2_ref[...], 0.0))
    h = _bfr(jnp.maximum(_dot(h, w3_ref[...]) + b3_ref[...], 0.0))
    out_ref[...] = _dot(h, w4_ref[...]) + b4_ref[...]


def _outmlp(addp, cnt, maxp, u, outw):
    return pl.pallas_call(
        _outmlp_body,
        out_shape=jax.ShapeDtypeStruct((G, 8), jnp.float32),
    )(addp, cnt, maxp, u, *outw)


def kernel(x, edge_attr, u, params, edge_index, batch):
    row = edge_index[0].astype(jnp.int32)
    col = edge_index[1].astype(jnp.int32)
    ea = edge_attr[:, 0].astype(jnp.float32)
    batch2 = batch.astype(jnp.int32).reshape(N, 1)

    L = params["layers"]

    def layer_w(p):
        wa = _bfr(p["We1"][:D])
        wb = _bfr(p["We1"][D:2 * D])
        wc = jnp.stack([_bfr(p["We1"][2 * D]), _bfr(p["We2"][:, 0]),
                        jnp.full((H,), p["be2"][0], jnp.float32)], axis=0)
        wn1a = _bfr(p["Wn1"][:D])
        wrow = _bfr(p["Wn1"][D:D + 3])
        wu = _bfr(p["Wn1"][D + 3:D + 4])
        bn1 = p["bn1"].reshape(1, H)
        bn2 = p["bn2"].reshape(1, D)
        ba = p["be1"].reshape(1, H)
        return wa, wb, wc, wn1a, wrow, wu, bn1, _bfr(p["Wn2"]), bn2, ba

    (wa0, wb0, wc0, wn1a0, wrow0, wu0, bn10, wn20, bn20, ba0) = layer_w(L[0])
    (wa1, wb1, wc1, wn1a1, wrow1, wu1, bn11, wn21, bn21, ba1) = layer_w(L[1])

    (W1, b1), (W2, b2), (W3, b3), (W4, b4) = params["out"]
    W1, W2, W3, W4 = _bfr(W1), _bfr(W2), _bfr(W3), _bfr(W4)
    w4p = jnp.zeros((H, 8), jnp.float32).at[:, :2].set(W4)
    b4p = jnp.zeros((1, 8), jnp.float32).at[0, :2].set(b4)
    outw = (W1[:D], W1[D:2 * D], W1[2 * D:3 * D], W1[3 * D:3 * D + 1],
            b1.reshape(1, H), W2, b2.reshape(1, H), W3, b3.reshape(1, H),
            w4p, b4p)

    def part3(p):
        return jnp.transpose(p.reshape(NW, NB, BN), (1, 0, 2))

    A, B = _prep(x, wa0, wb0, ba0)
    ps, pm, pc, se = _edge_call(A, B, row, col, ea, wc0)
    x1, A2, B2 = _node_fused(x, part3(ps), part3(pm), part3(pc), batch2, u,
                             wn1a0, wrow0, wu0, bn10, wn20, bn20,
                             wa1, wb1, ba1)
    ps, pm, pc, _ = _edge_call(A2, B2, row, col, se, wc1)
    x2 = _node_plain(x1, part3(ps), part3(pm), part3(pc), batch2, u,
                     wn1a1, wrow1, wu1, bn11, wn21, bn21)
    addp, cnt, maxp = _pool(x2, batch2)
    out = _outmlp(addp, cnt, maxp, u, outw)
    return out[:, :2]


# speedup vs baseline: 6.2108x; 1.9508x over previous
import functools

import jax
import jax.numpy as jnp
from jax import lax
from jax.experimental import pallas as pl
from jax.experimental.pallas import tpu as pltpu
from jax.experimental.pallas import tpu_sc as plsc

N = 10000
E = 320000
G = 16
D = 128
H = 128

NC = 2
NS = 16
NW = NC * NS
EPW = E // NW
CG = 80
NCHUNK = EPW // CG
NGRP = CG // 16
BINBLK = 2000

_F32MIN = -3.4e38


def _bfr(a):
    ui = lax.bitcast_convert_type(a, jnp.int32)
    bias = 0x7FFF + ((ui >> 16) & 1)
    r = (ui + bias) & ~0xFFFF
    return lax.bitcast_convert_type(r, jnp.float32)


def _vgather(v, idx):
    return lax.gather(
        v, idx[:, None],
        lax.GatherDimensionNumbers(
            offset_dims=(), collapsed_slice_dims=(0,), start_index_map=(0,)),
        slice_sizes=(1,), mode=lax.GatherScatterMode.PROMISE_IN_BOUNDS)


def _edge_body(a_hbm, b_hbm, row_hbm, col_hbm, ea_hbm, wc_hbm,
               sum_hbm, max_hbm, cnt_hbm, se_hbm,
               sumbin, maxbin, cntbin, rall, call_, eall, seall,
               abufa, bbufa, abufb, bbufb, wcbuf,
               semaa, semab, semba, sembb):
    wid = lax.axis_index("s") * NC + lax.axis_index("c")
    ebase = wid * EPW

    iota16 = lax.iota(jnp.int32, 16)
    zero16 = jnp.zeros((16,), jnp.float32)

    def init_body(i, _):
        sumbin[pl.ds(i * 16, 16)] = zero16
        cntbin[pl.ds(i * 16, 16)] = zero16
        maxbin[pl.ds(i * 16, 16)] = jnp.full((16,), _F32MIN, jnp.float32)
        return 0
    lax.fori_loop(0, N // 16, init_body, 0)

    pltpu.sync_copy(row_hbm.at[pl.ds(ebase, EPW)], rall)
    pltpu.sync_copy(col_hbm.at[pl.ds(ebase, EPW)], call_)
    pltpu.sync_copy(ea_hbm.at[pl.ds(ebase, EPW)], eall)

    pltpu.sync_copy(wc_hbm, wcbuf)
    w3 = [wcbuf[0, pl.ds(k * 16, 16)] for k in range(8)]
    w2 = [wcbuf[1, pl.ds(k * 16, 16)] for k in range(8)]
    be2v = wcbuf[2, pl.ds(0, 16)]

    def issue(ci, abuf, bbuf, sa, sb):
        pltpu.async_copy(a_hbm.at[rall.at[pl.ds(ci * CG, CG)]], abuf, sa)
        pltpu.async_copy(b_hbm.at[call_.at[pl.ds(ci * CG, CG)]], bbuf, sb)

    def wait(ci, abuf, bbuf, sa, sb):
        pltpu.make_async_copy(a_hbm.at[rall.at[pl.ds(ci * CG, CG)]],
                              abuf, sa).wait()
        pltpu.make_async_copy(b_hbm.at[call_.at[pl.ds(ci * CG, CG)]],
                              bbuf, sb).wait()

    def compute(ci, abuf, bbuf):
        def group_body(g, _):
            ebias = ci * CG + g * 16
            cvec = call_[pl.ds(ebias, 16)]
            evec = _bfr(eall[pl.ds(ebias, 16)])
            sv = zero16
            for e in range(16):
                eidx = g * 16 + e
                eav = _vgather(evec, jnp.full((16,), e, jnp.int32))
                acc = zero16
                for k in range(8):
                    va = abuf[eidx, pl.ds(k * 16, 16)]
                    vb = bbuf[eidx, pl.ds(k * 16, 16)]
                    t = va + vb + eav * w3[k]
                    t = _bfr(jnp.maximum(t, 0.0))
                    acc = acc + t * w2[k]
                for d in (8, 4, 2, 1):
                    acc = acc + _vgather(acc, iota16 ^ d)
                sv = jnp.where(iota16 == e, acc, sv)
            sv = sv + be2v
            seall[pl.ds(ebias, 16)] = sv

            sums = sv
            maxs = sv
            cnts = jnp.ones((16,), jnp.float32)
            bad = jnp.zeros((16,), jnp.bool_)
            for d in range(1, 16):
                idx = (iota16 + d) & 15
                same = cvec == _vgather(cvec, idx)
                svg = _vgather(sv, idx)
                wrap = iota16 >= (16 - d)
                bad = bad | (same & wrap)
                sums = sums + jnp.where(same, svg, 0.0)
                cnts = cnts + jnp.where(same, 1.0, 0.0)
                maxs = jnp.where(same, jnp.maximum(maxs, svg), maxs)
            first = ~bad
            plsc.addupdate_scatter(sumbin, [cvec], sums, mask=first)
            plsc.addupdate_scatter(cntbin, [cvec], cnts, mask=first)
            old = plsc.load_gather(maxbin, [cvec])
            plsc.store_scatter(maxbin, [cvec], jnp.maximum(old, maxs),
                               mask=first)
            return 0

        lax.fori_loop(0, NGRP, group_body, 0)

    issue(0, abufa, bbufa, semaa, semab)

    def pair_body(g, _):
        c0 = 2 * g
        c1 = c0 + 1

        @pl.when(c1 < NCHUNK)
        def _():
            issue(c1, abufb, bbufb, semba, sembb)

        wait(c0, abufa, bbufa, semaa, semab)
        compute(c0, abufa, bbufa)

        @pl.when(c1 < NCHUNK)
        def _():
            @pl.when(c1 + 1 < NCHUNK)
            def _():
                issue(c1 + 1, abufa, bbufa, semaa, semab)

            wait(c1, abufb, bbufb, semba, sembb)
            compute(c1, abufb, bbufb)

        return 0

    lax.fori_loop(0, (NCHUNK + 1) // 2, pair_body, 0)

    pltpu.sync_copy(seall, se_hbm.at[pl.ds(ebase, EPW)])
    pltpu.sync_copy(sumbin, sum_hbm.at[wid])
    pltpu.sync_copy(maxbin, max_hbm.at[wid])
    pltpu.sync_copy(cntbin, cnt_hbm.at[wid])


_edge_call = pl.kernel(
    _edge_body,
    out_type=(
        jax.ShapeDtypeStruct((NW, N), jnp.float32),
        jax.ShapeDtypeStruct((NW, N), jnp.float32),
        jax.ShapeDtypeStruct((NW, N), jnp.float32),
        jax.ShapeDtypeStruct((E,), jnp.float32),
    ),
    mesh=plsc.VectorSubcoreMesh(core_axis_name="c", subcore_axis_name="s"),
    compiler_params=pltpu.CompilerParams(needs_layout_passes=False),
    scratch_types=[
        pltpu.VMEM((N,), jnp.float32),
        pltpu.VMEM((N,), jnp.float32),
        pltpu.VMEM((N,), jnp.float32),
        pltpu.VMEM((EPW,), jnp.int32),
        pltpu.VMEM((EPW,), jnp.int32),
        pltpu.VMEM((EPW,), jnp.float32),
        pltpu.VMEM((EPW,), jnp.float32),
        pltpu.VMEM((CG, H), jnp.float32),
        pltpu.VMEM((CG, H), jnp.float32),
        pltpu.VMEM((CG, H), jnp.float32),
        pltpu.VMEM((CG, H), jnp.float32),
        pltpu.VMEM((3, H), jnp.float32),
        pltpu.SemaphoreType.DMA,
        pltpu.SemaphoreType.DMA,
        pltpu.SemaphoreType.DMA,
        pltpu.SemaphoreType.DMA,
    ],
)


BN = 2000
NB = N // BN
_HI = lax.Precision.HIGHEST


def _dot(a, b):
    return jnp.dot(a, b, preferred_element_type=jnp.float32, precision=_HI)


def _dot0(a, b):
    return lax.dot_general(a, b, (((0,), (0,)), ((), ())),
                           preferred_element_type=jnp.float32, precision=_HI)


def _full_spec(shape):
    return pl.BlockSpec(shape, lambda i: tuple(0 for _ in shape))


def _prep_body(x_ref, wa_ref, wb_ref, ba_ref, a_ref, b_ref):
    xv = _bfr(x_ref[...])
    a_ref[...] = _dot(xv, wa_ref[...]) + ba_ref[...]
    b_ref[...] = _dot(xv, wb_ref[...])


def _prep(x, wa, wb, ba):
    row_spec = pl.BlockSpec((BN, D), lambda i: (i, 0))
    return pl.pallas_call(
        _prep_body,
        grid=(NB,),
        in_specs=[row_spec, _full_spec((D, D)), _full_spec((D, D)),
                  _full_spec((1, H))],
        out_specs=(row_spec, row_spec),
        out_shape=(jax.ShapeDtypeStruct((N, D), jnp.float32),
                   jax.ShapeDtypeStruct((N, D), jnp.float32)),
    )(x, wa, wb, ba)


def _node_common(x_ref, ps_ref, pm_ref, pc_ref, batch_ref, u_ref,
                 wn1a_ref, wrow_ref, wu_ref, bn1_ref, wn2_ref, bn2_ref):
    s = jnp.sum(ps_ref[0], axis=0, keepdims=True)
    c = jnp.sum(pc_ref[0], axis=0, keepdims=True)
    m = jnp.max(pm_ref[0], axis=0, keepdims=True)
    m = jnp.where(c > 0.0, m, 0.0)
    mean = s / jnp.maximum(c, 1.0)

    wrow = wrow_ref[...]
    onehot = (batch_ref[...] ==
              lax.broadcasted_iota(jnp.int32, (BN, G), 1)).astype(jnp.float32)
    uw = _dot(_bfr(u_ref[...]), wu_ref[...])

    h = (_dot(_bfr(x_ref[...]), wn1a_ref[...])
         + _dot0(_bfr(s), wrow[0:1])
         + _dot0(_bfr(m), wrow[1:2])
         + _dot0(_bfr(mean), wrow[2:3])
         + _dot(onehot, uw)
         + bn1_ref[...])
    h = _bfr(jnp.maximum(h, 0.0))
    return _dot(h, wn2_ref[...]) + bn2_ref[...]


def _node_fused_body(x_ref, ps_ref, pm_ref, pc_ref, batch_ref, u_ref,
                     wn1a_ref, wrow_ref, wu_ref, bn1_ref, wn2_ref, bn2_ref,
                     wea_ref, web_ref, bea_ref,
                     xn_ref, a2_ref, b2_ref):
    xn = _node_common(x_ref, ps_ref, pm_ref, pc_ref, batch_ref, u_ref,
                      wn1a_ref, wrow_ref, wu_ref, bn1_ref, wn2_ref, bn2_ref)
    xn_ref[...] = xn
    xnr = _bfr(xn)
    a2_ref[...] = _dot(xnr, wea_ref[...]) + bea_ref[...]
    b2_ref[...] = _dot(xnr, web_ref[...])


def _node_plain_body(x_ref, ps_ref, pm_ref, pc_ref, batch_ref, u_ref,
                     wn1a_ref, wrow_ref, wu_ref, bn1_ref, wn2_ref, bn2_ref,
                     xn_ref):
    xn_ref[...] = _node_common(x_ref, ps_ref, pm_ref, pc_ref, batch_ref,
                               u_ref, wn1a_ref, wrow_ref, wu_ref, bn1_ref,
                               wn2_ref, bn2_ref)


def _node_specs():
    row_spec = pl.BlockSpec((BN, D), lambda i: (i, 0))
    part_spec = pl.BlockSpec((1, NW, BN), lambda i: (i, 0, 0))
    batch_spec = pl.BlockSpec((BN, 1), lambda i: (i, 0))
    wspecs = [_full_spec((G, 1)), _full_spec((D, D)), _full_spec((3, H)),
              _full_spec((1, H)), _full_spec((1, H)), _full_spec((D, D)),
              _full_spec((1, D))]
    return [row_spec, part_spec, part_spec, part_spec, batch_spec] + wspecs


def _node_fused(x, ps, pm, pc, batch2, u, wn1a, wrow, wu, bn1, wn2, bn2,
                wea, web, bea):
    row_spec = pl.BlockSpec((BN, D), lambda i: (i, 0))
    return pl.pallas_call(
        _node_fused_body,
        grid=(NB,),
        in_specs=_node_specs() + [_full_spec((D, D)), _full_spec((D, D)),
                                  _full_spec((1, H))],
        out_specs=(row_spec, row_spec, row_spec),
        out_shape=(jax.ShapeDtypeStruct((N, D), jnp.float32),
                   jax.ShapeDtypeStruct((N, D), jnp.float32),
                   jax.ShapeDtypeStruct((N, D), jnp.float32)),
    )(x, ps, pm, pc, batch2, u, wn1a, wrow, wu, bn1, wn2, bn2, wea, web, bea)


def _node_plain(x, ps, pm, pc, batch2, u, wn1a, wrow, wu, bn1, wn2, bn2):
    row_spec = pl.BlockSpec((BN, D), lambda i: (i, 0))
    return pl.pallas_call(
        _node_plain_body,
        grid=(NB,),
        in_specs=_node_specs(),
        out_specs=row_spec,
        out_shape=jax.ShapeDtypeStruct((N, D), jnp.float32),
    )(x, ps, pm, pc, batch2, u, wn1a, wrow, wu, bn1, wn2, bn2)


def _pool_body(x_ref, batch_ref, addp_ref, cnt_ref, maxp_ref):
    i = pl.program_id(0)
    xv = x_ref[...]
    bv = batch_ref[...]
    onehot = (bv == lax.broadcasted_iota(jnp.int32, (BN, G), 1)
              ).astype(jnp.float32)
    addp = _dot0(onehot, xv)
    cnt = _dot0(onehot, jnp.ones((BN, 1), jnp.float32))
    rows = []
    for g in range(G):
        rows.append(jnp.max(jnp.where(bv == g, xv, _F32MIN), axis=0,
                            keepdims=True))
    maxp = jnp.concatenate(rows, axis=0)

    @pl.when(i == 0)
    def _():
        addp_ref[...] = jnp.zeros((G, D), jnp.float32)
        cnt_ref[...] = jnp.zeros((G, 1), jnp.float32)
        maxp_ref[...] = jnp.full((G, D), _F32MIN, jnp.float32)

    addp_ref[...] += addp
    cnt_ref[...] += cnt
    maxp_ref[...] = jnp.maximum(maxp_ref[...], maxp)


def _pool(x, batch2):
    return pl.pallas_call(
        _pool_body,
        grid=(NB,),
        in_specs=[pl.BlockSpec((BN, D), lambda i: (i, 0)),
                  pl.BlockSpec((BN, 1), lambda i: (i, 0))],
        out_specs=(_full_spec((G, D)), _full_spec((G, 1)),
                   _full_spec((G, D))),
        out_shape=(jax.ShapeDtypeStruct((G, D), jnp.float32),
                   jax.ShapeDtypeStruct((G, 1), jnp.float32),
                   jax.ShapeDtypeStruct((G, D), jnp.float32)),
    )(x, batch2)


def _outmlp_body(addp_ref, cnt_ref, maxp_ref, u_ref,
                 w1a_ref, w1b_ref, w1c_ref, w1u_ref, b1_ref,
                 w2_ref, b2_ref, w3_ref, b3_ref, w4_ref, b4_ref, out_ref):
    addp = addp_ref[...]
    cnt = cnt_ref[...]
    maxp = jnp.where(cnt > 0.0, maxp_ref[...], 0.0)
    meanp = addp / jnp.maximum(cnt, 1.0)
    h = (_dot(_bfr(addp), w1a_ref[...]) + _dot(_bfr(meanp), w1b_ref[...])
         + _dot(_bfr(maxp), w1c_ref[...]) + _dot(_bfr(u_ref[...]),
                                                 w1u_ref[...])
         + b1_ref[...])
    h = _bfr(jnp.maximum(h, 0.0))
    h = _bfr(jnp.maximum(_dot(h, w2_ref[...]) + b2_ref[...], 0.0))
    h = _bfr(jnp.maximum(_dot(h, w3_ref[...]) + b3_ref[...], 0.0))
    out_ref[...] = _dot(h, w4_ref[...]) + b4_ref[...]


def _outmlp(addp, cnt, maxp, u, outw):
    return pl.pallas_call(
        _outmlp_body,
        out_shape=jax.ShapeDtypeStruct((G, 8), jnp.float32),
    )(addp, cnt, maxp, u, *outw)


def kernel(x, edge_attr, u, params, edge_index, batch):
    row = edge_index[0].astype(jnp.int32)
    col = edge_index[1].astype(jnp.int32)
    ea = edge_attr[:, 0].astype(jnp.float32)
    batch2 = batch.astype(jnp.int32).reshape(N, 1)

    L = params["layers"]

    def layer_w(p):
        wa = _bfr(p["We1"][:D])
        wb = _bfr(p["We1"][D:2 * D])
        wc = jnp.stack([_bfr(p["We1"][2 * D]), _bfr(p["We2"][:, 0]),
                        jnp.full((H,), p["be2"][0], jnp.float32)], axis=0)
        wn1a = _bfr(p["Wn1"][:D])
        wrow = _bfr(p["Wn1"][D:D + 3])
        wu = _bfr(p["Wn1"][D + 3:D + 4])
        bn1 = p["bn1"].reshape(1, H)
        bn2 = p["bn2"].reshape(1, D)
        ba = p["be1"].reshape(1, H)
        return wa, wb, wc, wn1a, wrow, wu, bn1, _bfr(p["Wn2"]), bn2, ba

    (wa0, wb0, wc0, wn1a0, wrow0, wu0, bn10, wn20, bn20, ba0) = layer_w(L[0])
    (wa1, wb1, wc1, wn1a1, wrow1, wu1, bn11, wn21, bn21, ba1) = layer_w(L[1])

    (W1, b1), (W2, b2), (W3, b3), (W4, b4) = params["out"]
    W1, W2, W3, W4 = _bfr(W1), _bfr(W2), _bfr(W3), _bfr(W4)
    w4p = jnp.zeros((H, 8), jnp.float32).at[:, :2].set(W4)
    b4p = jnp.zeros((1, 8), jnp.float32).at[0, :2].set(b4)
    outw = (W1[:D], W1[D:2 * D], W1[2 * D:3 * D], W1[3 * D:3 * D + 1],
            b1.reshape(1, H), W2, b2.reshape(1, H), W3, b3.reshape(1, H),
            w4p, b4p)

    def part3(p):
        return jnp.transpose(p.reshape(NW, NB, BN), (1, 0, 2))

    A, B = _prep(x, wa0, wb0, ba0)
    ps, pm, pc, se = _edge_call(A, B, row, col, ea, wc0)
    x1, A2, B2 = _node_fused(x, part3(ps), part3(pm), part3(pc), batch2, u,
                             wn1a0, wrow0, wu0, bn10, wn20, bn20,
                             wa1, wb1, ba1)
    ps, pm, pc, _ = _edge_call(A2, B2, row, col, se, wc1)
    x2 = _node_plain(x1, part3(ps), part3(pm), part3(pc), batch2, u,
                     wn1a1, wrow1, wu1, bn11, wn21, bn21)
    addp, cnt, maxp = _pool(x2, batch2)
    out = _outmlp(addp, cnt, maxp, u, outw)
    return out[:, :2]


# speedup vs baseline: 6.2277x; 1.0027x over previous
import functools

import jax
import jax.numpy as jnp
from jax import lax
from jax.experimental import pallas as pl
from jax.experimental.pallas import tpu as pltpu
from jax.experimental.pallas import tpu_sc as plsc

N = 10000
E = 320000
G = 16
D = 128
H = 128

NC = 2
NS = 16
NW = NC * NS
EPW = E // NW
CG = 80
NCHUNK = EPW // CG
NGRP = CG // 16
BINBLK = 2000

_F32MIN = -3.4e38


def _bfr(a):
    ui = lax.bitcast_convert_type(a, jnp.int32)
    bias = 0x7FFF + ((ui >> 16) & 1)
    r = (ui + bias) & ~0xFFFF
    return lax.bitcast_convert_type(r, jnp.float32)


def _vgather(v, idx):
    return lax.gather(
        v, idx[:, None],
        lax.GatherDimensionNumbers(
            offset_dims=(), collapsed_slice_dims=(0,), start_index_map=(0,)),
        slice_sizes=(1,), mode=lax.GatherScatterMode.PROMISE_IN_BOUNDS)


def _edge_body(a_hbm, b_hbm, row_hbm, col_hbm, ea_hbm, wc_hbm,
               sum_hbm, max_hbm, cnt_hbm, se_hbm,
               sumbin, maxbin, cntbin, rall, call_, eall, seall,
               abufa, bbufa, abufb, bbufb, wcbuf,
               semaa, semab, semba, sembb):
    wid = lax.axis_index("s") * NC + lax.axis_index("c")
    ebase = wid * EPW

    iota16 = lax.iota(jnp.int32, 16)
    zero16 = jnp.zeros((16,), jnp.float32)

    def init_body(i, _):
        sumbin[pl.ds(i * 16, 16)] = zero16
        cntbin[pl.ds(i * 16, 16)] = zero16
        maxbin[pl.ds(i * 16, 16)] = jnp.full((16,), _F32MIN, jnp.float32)
        return 0
    lax.fori_loop(0, N // 16, init_body, 0)

    pltpu.sync_copy(row_hbm.at[pl.ds(ebase, EPW)], rall)
    pltpu.sync_copy(col_hbm.at[pl.ds(ebase, EPW)], call_)
    pltpu.sync_copy(ea_hbm.at[pl.ds(ebase, EPW)], eall)

    pltpu.sync_copy(wc_hbm, wcbuf)
    w3 = [wcbuf[0, pl.ds(k * 16, 16)] for k in range(8)]
    w2 = [wcbuf[1, pl.ds(k * 16, 16)] for k in range(8)]
    be2v = wcbuf[2, pl.ds(0, 16)]

    def issue(ci, abuf, bbuf, sa, sb):
        pltpu.async_copy(a_hbm.at[rall.at[pl.ds(ci * CG, CG)]], abuf, sa)
        pltpu.async_copy(b_hbm.at[call_.at[pl.ds(ci * CG, CG)]], bbuf, sb)

    def wait(ci, abuf, bbuf, sa, sb):
        pltpu.make_async_copy(a_hbm.at[rall.at[pl.ds(ci * CG, CG)]],
                              abuf, sa).wait()
        pltpu.make_async_copy(b_hbm.at[call_.at[pl.ds(ci * CG, CG)]],
                              bbuf, sb).wait()

    def compute(ci, abuf, bbuf):
        def group_body(g, _):
            ebias = ci * CG + g * 16
            cvec = call_[pl.ds(ebias, 16)]
            evec = _bfr(eall[pl.ds(ebias, 16)])
            sv = zero16
            for e in range(16):
                eidx = g * 16 + e
                eav = _vgather(evec, jnp.full((16,), e, jnp.int32))
                ts = []
                for k in range(8):
                    va = abuf[eidx, pl.ds(k * 16, 16)]
                    vb = bbuf[eidx, pl.ds(k * 16, 16)]
                    ts.append(jnp.maximum(va + vb + eav * w3[k], 0.0))
                ps = [_bfr(ts[k]) * w2[k] for k in range(8)]
                acc = ((ps[0] + ps[1]) + (ps[2] + ps[3])) + (
                    (ps[4] + ps[5]) + (ps[6] + ps[7]))
                for d in (8, 4, 2, 1):
                    acc = acc + _vgather(acc, iota16 ^ d)
                sv = jnp.where(iota16 == e, acc, sv)
            sv = sv + be2v
            seall[pl.ds(ebias, 16)] = sv

            sums = sv
            maxs = sv
            cnts = jnp.ones((16,), jnp.float32)
            bad = jnp.zeros((16,), jnp.bool_)
            for d in range(1, 16):
                idx = (iota16 + d) & 15
                same = cvec == _vgather(cvec, idx)
                svg = _vgather(sv, idx)
                wrap = iota16 >= (16 - d)
                bad = bad | (same & wrap)
                sums = sums + jnp.where(same, svg, 0.0)
                cnts = cnts + jnp.where(same, 1.0, 0.0)
                maxs = jnp.where(same, jnp.maximum(maxs, svg), maxs)
            first = ~bad
            plsc.addupdate_scatter(sumbin, [cvec], sums, mask=first)
            plsc.addupdate_scatter(cntbin, [cvec], cnts, mask=first)
            old = plsc.load_gather(maxbin, [cvec])
            plsc.store_scatter(maxbin, [cvec], jnp.maximum(old, maxs),
                               mask=first)
            return 0

        lax.fori_loop(0, NGRP, group_body, 0)

    issue(0, abufa, bbufa, semaa, semab)

    def pair_body(g, _):
        c0 = 2 * g
        c1 = c0 + 1

        @pl.when(c1 < NCHUNK)
        def _():
            issue(c1, abufb, bbufb, semba, sembb)

        wait(c0, abufa, bbufa, semaa, semab)
        compute(c0, abufa, bbufa)

        @pl.when(c1 < NCHUNK)
        def _():
            @pl.when(c1 + 1 < NCHUNK)
            def _():
                issue(c1 + 1, abufa, bbufa, semaa, semab)

            wait(c1, abufb, bbufb, semba, sembb)
            compute(c1, abufb, bbufb)

        return 0

    lax.fori_loop(0, (NCHUNK + 1) // 2, pair_body, 0)

    pltpu.sync_copy(seall, se_hbm.at[pl.ds(ebase, EPW)])
    pltpu.sync_copy(sumbin, sum_hbm.at[wid])
    pltpu.sync_copy(maxbin, max_hbm.at[wid])
    pltpu.sync_copy(cntbin, cnt_hbm.at[wid])


_edge_call = pl.kernel(
    _edge_body,
    out_type=(
        jax.ShapeDtypeStruct((NW, N), jnp.float32),
        jax.ShapeDtypeStruct((NW, N), jnp.float32),
        jax.ShapeDtypeStruct((NW, N), jnp.float32),
        jax.ShapeDtypeStruct((E,), jnp.float32),
    ),
    mesh=plsc.VectorSubcoreMesh(core_axis_name="c", subcore_axis_name="s"),
    compiler_params=pltpu.CompilerParams(needs_layout_passes=False),
    scratch_types=[
        pltpu.VMEM((N,), jnp.float32),
        pltpu.VMEM((N,), jnp.float32),
        pltpu.VMEM((N,), jnp.float32),
        pltpu.VMEM((EPW,), jnp.int32),
        pltpu.VMEM((EPW,), jnp.int32),
        pltpu.VMEM((EPW,), jnp.float32),
        pltpu.VMEM((EPW,), jnp.float32),
        pltpu.VMEM((CG, H), jnp.float32),
        pltpu.VMEM((CG, H), jnp.float32),
        pltpu.VMEM((CG, H), jnp.float32),
        pltpu.VMEM((CG, H), jnp.float32),
        pltpu.VMEM((3, H), jnp.float32),
        pltpu.SemaphoreType.DMA,
        pltpu.SemaphoreType.DMA,
        pltpu.SemaphoreType.DMA,
        pltpu.SemaphoreType.DMA,
    ],
)


BN = 2000
NB = N // BN
_HI = lax.Precision.HIGHEST


def _dot(a, b):
    return jnp.dot(a, b, preferred_element_type=jnp.float32, precision=_HI)


def _dot0(a, b):
    return lax.dot_general(a, b, (((0,), (0,)), ((), ())),
                           preferred_element_type=jnp.float32, precision=_HI)


def _full_spec(shape):
    return pl.BlockSpec(shape, lambda i: tuple(0 for _ in shape))


def _prep_body(x_ref, wa_ref, wb_ref, ba_ref, a_ref, b_ref):
    xv = _bfr(x_ref[...])
    a_ref[...] = _dot(xv, wa_ref[...]) + ba_ref[...]
    b_ref[...] = _dot(xv, wb_ref[...])


def _prep(x, wa, wb, ba):
    row_spec = pl.BlockSpec((BN, D), lambda i: (i, 0))
    return pl.pallas_call(
        _prep_body,
        grid=(NB,),
        in_specs=[row_spec, _full_spec((D, D)), _full_spec((D, D)),
                  _full_spec((1, H))],
        out_specs=(row_spec, row_spec),
        out_shape=(jax.ShapeDtypeStruct((N, D), jnp.float32),
                   jax.ShapeDtypeStruct((N, D), jnp.float32)),
    )(x, wa, wb, ba)


def _node_common(x_ref, ps_ref, pm_ref, pc_ref, batch_ref, u_ref,
                 wn1a_ref, wrow_ref, wu_ref, bn1_ref, wn2_ref, bn2_ref):
    s = jnp.sum(ps_ref[0], axis=0, keepdims=True)
    c = jnp.sum(pc_ref[0], axis=0, keepdims=True)
    m = jnp.max(pm_ref[0], axis=0, keepdims=True)
    m = jnp.where(c > 0.0, m, 0.0)
    mean = s / jnp.maximum(c, 1.0)

    wrow = wrow_ref[...]
    onehot = (batch_ref[...] ==
              lax.broadcasted_iota(jnp.int32, (BN, G), 1)).astype(jnp.float32)
    uw = _dot(_bfr(u_ref[...]), wu_ref[...])

    h = (_dot(_bfr(x_ref[...]), wn1a_ref[...])
         + _dot0(_bfr(s), wrow[0:1])
         + _dot0(_bfr(m), wrow[1:2])
         + _dot0(_bfr(mean), wrow[2:3])
         + _dot(onehot, uw)
         + bn1_ref[...])
    h = _bfr(jnp.maximum(h, 0.0))
    return _dot(h, wn2_ref[...]) + bn2_ref[...]


def _node_fused_body(x_ref, ps_ref, pm_ref, pc_ref, batch_ref, u_ref,
                     wn1a_ref, wrow_ref, wu_ref, bn1_ref, wn2_ref, bn2_ref,
                     wea_ref, web_ref, bea_ref,
                     xn_ref, a2_ref, b2_ref):
    xn = _node_common(x_ref, ps_ref, pm_ref, pc_ref, batch_ref, u_ref,
                      wn1a_ref, wrow_ref, wu_ref, bn1_ref, wn2_ref, bn2_ref)
    xn_ref[...] = xn
    xnr = _bfr(xn)
    a2_ref[...] = _dot(xnr, wea_ref[...]) + bea_ref[...]
    b2_ref[...] = _dot(xnr, web_ref[...])


def _node_plain_body(x_ref, ps_ref, pm_ref, pc_ref, batch_ref, u_ref,
                     wn1a_ref, wrow_ref, wu_ref, bn1_ref, wn2_ref, bn2_ref,
                     xn_ref):
    xn_ref[...] = _node_common(x_ref, ps_ref, pm_ref, pc_ref, batch_ref,
                               u_ref, wn1a_ref, wrow_ref, wu_ref, bn1_ref,
                               wn2_ref, bn2_ref)


def _node_specs():
    row_spec = pl.BlockSpec((BN, D), lambda i: (i, 0))
    part_spec = pl.BlockSpec((1, NW, BN), lambda i: (i, 0, 0))
    batch_spec = pl.BlockSpec((BN, 1), lambda i: (i, 0))
    wspecs = [_full_spec((G, 1)), _full_spec((D, D)), _full_spec((3, H)),
              _full_spec((1, H)), _full_spec((1, H)), _full_spec((D, D)),
              _full_spec((1, D))]
    return [row_spec, part_spec, part_spec, part_spec, batch_spec] + wspecs


def _node_fused(x, ps, pm, pc, batch2, u, wn1a, wrow, wu, bn1, wn2, bn2,
                wea, web, bea):
    row_spec = pl.BlockSpec((BN, D), lambda i: (i, 0))
    return pl.pallas_call(
        _node_fused_body,
        grid=(NB,),
        in_specs=_node_specs() + [_full_spec((D, D)), _full_spec((D, D)),
                                  _full_spec((1, H))],
        out_specs=(row_spec, row_spec, row_spec),
        out_shape=(jax.ShapeDtypeStruct((N, D), jnp.float32),
                   jax.ShapeDtypeStruct((N, D), jnp.float32),
                   jax.ShapeDtypeStruct((N, D), jnp.float32)),
    )(x, ps, pm, pc, batch2, u, wn1a, wrow, wu, bn1, wn2, bn2, wea, web, bea)


def _node_plain(x, ps, pm, pc, batch2, u, wn1a, wrow, wu, bn1, wn2, bn2):
    row_spec = pl.BlockSpec((BN, D), lambda i: (i, 0))
    return pl.pallas_call(
        _node_plain_body,
        grid=(NB,),
        in_specs=_node_specs(),
        out_specs=row_spec,
        out_shape=jax.ShapeDtypeStruct((N, D), jnp.float32),
    )(x, ps, pm, pc, batch2, u, wn1a, wrow, wu, bn1, wn2, bn2)


def _pool_body(x_ref, batch_ref, addp_ref, cnt_ref, maxp_ref):
    i = pl.program_id(0)
    xv = x_ref[...]
    bv = batch_ref[...]
    onehot = (bv == lax.broadcasted_iota(jnp.int32, (BN, G), 1)
              ).astype(jnp.float32)
    addp = _dot0(onehot, xv)
    cnt = _dot0(onehot, jnp.ones((BN, 1), jnp.float32))
    rows = []
    for g in range(G):
        rows.append(jnp.max(jnp.where(bv == g, xv, _F32MIN), axis=0,
                            keepdims=True))
    maxp = jnp.concatenate(rows, axis=0)

    @pl.when(i == 0)
    def _():
        addp_ref[...] = jnp.zeros((G, D), jnp.float32)
        cnt_ref[...] = jnp.zeros((G, 1), jnp.float32)
        maxp_ref[...] = jnp.full((G, D), _F32MIN, jnp.float32)

    addp_ref[...] += addp
    cnt_ref[...] += cnt
    maxp_ref[...] = jnp.maximum(maxp_ref[...], maxp)


def _pool(x, batch2):
    return pl.pallas_call(
        _pool_body,
        grid=(NB,),
        in_specs=[pl.BlockSpec((BN, D), lambda i: (i, 0)),
                  pl.BlockSpec((BN, 1), lambda i: (i, 0))],
        out_specs=(_full_spec((G, D)), _full_spec((G, 1)),
                   _full_spec((G, D))),
        out_shape=(jax.ShapeDtypeStruct((G, D), jnp.float32),
                   jax.ShapeDtypeStruct((G, 1), jnp.float32),
                   jax.ShapeDtypeStruct((G, D), jnp.float32)),
    )(x, batch2)


def _outmlp_body(addp_ref, cnt_ref, maxp_ref, u_ref,
                 w1a_ref, w1b_ref, w1c_ref, w1u_ref, b1_ref,
                 w2_ref, b2_ref, w3_ref, b3_ref, w4_ref, b4_ref, out_ref):
    addp = addp_ref[...]
    cnt = cnt_ref[...]
    maxp = jnp.where(cnt > 0.0, maxp_ref[...], 0.0)
    meanp = addp / jnp.maximum(cnt, 1.0)
    h = (_dot(_bfr(addp), w1a_ref[...]) + _dot(_bfr(meanp), w1b_ref[...])
         + _dot(_bfr(maxp), w1c_ref[...]) + _dot(_bfr(u_ref[...]),
                                                 w1u_ref[...])
         + b1_ref[...])
    h = _bfr(jnp.maximum(h, 0.0))
    h = _bfr(jnp.maximum(_dot(h, w2_ref[...]) + b2_ref[...], 0.0))
    h = _bfr(jnp.maximum(_dot(h, w3_ref[...]) + b3_ref[...], 0.0))
    out_ref[...] = _dot(h, w4_ref[...]) + b4_ref[...]


def _outmlp(addp, cnt, maxp, u, outw):
    return pl.pallas_call(
        _outmlp_body,
        out_shape=jax.ShapeDtypeStruct((G, 8), jnp.float32),
    )(addp, cnt, maxp, u, *outw)


def kernel(x, edge_attr, u, params, edge_index, batch):
    row = edge_index[0].astype(jnp.int32)
    col = edge_index[1].astype(jnp.int32)
    ea = edge_attr[:, 0].astype(jnp.float32)
    batch2 = batch.astype(jnp.int32).reshape(N, 1)

    L = params["layers"]

    def layer_w(p):
        wa = _bfr(p["We1"][:D])
        wb = _bfr(p["We1"][D:2 * D])
        wc = jnp.stack([_bfr(p["We1"][2 * D]), _bfr(p["We2"][:, 0]),
                        jnp.full((H,), p["be2"][0], jnp.float32)], axis=0)
        wn1a = _bfr(p["Wn1"][:D])
        wrow = _bfr(p["Wn1"][D:D + 3])
        wu = _bfr(p["Wn1"][D + 3:D + 4])
        bn1 = p["bn1"].reshape(1, H)
        bn2 = p["bn2"].reshape(1, D)
        ba = p["be1"].reshape(1, H)
        return wa, wb, wc, wn1a, wrow, wu, bn1, _bfr(p["Wn2"]), bn2, ba

    (wa0, wb0, wc0, wn1a0, wrow0, wu0, bn10, wn20, bn20, ba0) = layer_w(L[0])
    (wa1, wb1, wc1, wn1a1, wrow1, wu1, bn11, wn21, bn21, ba1) = layer_w(L[1])

    (W1, b1), (W2, b2), (W3, b3), (W4, b4) = params["out"]
    W1, W2, W3, W4 = _bfr(W1), _bfr(W2), _bfr(W3), _bfr(W4)
    w4p = jnp.zeros((H, 8), jnp.float32).at[:, :2].set(W4)
    b4p = jnp.zeros((1, 8), jnp.float32).at[0, :2].set(b4)
    outw = (W1[:D], W1[D:2 * D], W1[2 * D:3 * D], W1[3 * D:3 * D + 1],
            b1.reshape(1, H), W2, b2.reshape(1, H), W3, b3.reshape(1, H),
            w4p, b4p)

    def part3(p):
        return jnp.transpose(p.reshape(NW, NB, BN), (1, 0, 2))

    A, B = _prep(x, wa0, wb0, ba0)
    ps, pm, pc, se = _edge_call(A, B, row, col, ea, wc0)
    x1, A2, B2 = _node_fused(x, part3(ps), part3(pm), part3(pc), batch2, u,
                             wn1a0, wrow0, wu0, bn10, wn20, bn20,
                             wa1, wb1, ba1)
    ps, pm, pc, _ = _edge_call(A2, B2, row, col, se, wc1)
    x2 = _node_plain(x1, part3(ps), part3(pm), part3(pc), batch2, u,
                     wn1a1, wrow1, wu1, bn11, wn21, bn21)
    addp, cnt, maxp = _pool(x2, batch2)
    out = _outmlp(addp, cnt, maxp, u, outw)
    return out[:, :2]


# speedup vs baseline: 7.1556x; 1.1490x over previous
import functools

import jax
import jax.numpy as jnp
from jax import lax
from jax.experimental import pallas as pl
from jax.experimental.pallas import tpu as pltpu
from jax.experimental.pallas import tpu_sc as plsc

N = 10000
E = 320000
G = 16
D = 128
H = 128

NC = 2
NS = 16
NW = NC * NS
EPW = E // NW
CG = 80
NCHUNK = EPW // CG
NGRP = CG // 16
BINBLK = 2000

_F32MIN = -3.4e38


def _bfr(a):
    ui = lax.bitcast_convert_type(a, jnp.int32)
    bias = 0x7FFF + ((ui >> 16) & 1)
    r = (ui + bias) & ~0xFFFF
    return lax.bitcast_convert_type(r, jnp.float32)


def _bfr2(a):
    ui = lax.bitcast_convert_type(a, jnp.int32)
    r = (ui + 0x8000) & ~0xFFFF
    return lax.bitcast_convert_type(r, jnp.float32)


def _vgather(v, idx):
    return lax.gather(
        v, idx[:, None],
        lax.GatherDimensionNumbers(
            offset_dims=(), collapsed_slice_dims=(0,), start_index_map=(0,)),
        slice_sizes=(1,), mode=lax.GatherScatterMode.PROMISE_IN_BOUNDS)


def _edge_body(a_hbm, b_hbm, row_hbm, col_hbm, ea_hbm, wc_hbm,
               sum_hbm, max_hbm, cnt_hbm, se_hbm,
               sumbin, maxbin, cntbin, rall, call_, eall, seall,
               abufa, bbufa, abufb, bbufb, wcbuf,
               semaa, semab, semba, sembb):
    wid = lax.axis_index("s") * NC + lax.axis_index("c")
    ebase = wid * EPW

    iota16 = lax.iota(jnp.int32, 16)
    zero16 = jnp.zeros((16,), jnp.float32)

    def init_body(i, _):
        sumbin[pl.ds(i * 16, 16)] = zero16
        cntbin[pl.ds(i * 16, 16)] = zero16
        maxbin[pl.ds(i * 16, 16)] = jnp.full((16,), _F32MIN, jnp.float32)
        return 0
    lax.fori_loop(0, N // 16, init_body, 0)

    pltpu.sync_copy(row_hbm.at[pl.ds(ebase, EPW)], rall)
    pltpu.sync_copy(col_hbm.at[pl.ds(ebase, EPW)], call_)
    pltpu.sync_copy(ea_hbm.at[pl.ds(ebase, EPW)], eall)

    pltpu.sync_copy(wc_hbm, wcbuf)
    w3 = [wcbuf[0, pl.ds(k * 16, 16)] for k in range(8)]
    w2 = [wcbuf[1, pl.ds(k * 16, 16)] for k in range(8)]
    be2v = wcbuf[2, pl.ds(0, 16)]

    def issue(ci, abuf, bbuf, sa, sb):
        pltpu.async_copy(a_hbm.at[rall.at[pl.ds(ci * CG, CG)]], abuf, sa)
        pltpu.async_copy(b_hbm.at[call_.at[pl.ds(ci * CG, CG)]], bbuf, sb)

    def wait(ci, abuf, bbuf, sa, sb):
        pltpu.make_async_copy(a_hbm.at[rall.at[pl.ds(ci * CG, CG)]],
                              abuf, sa).wait()
        pltpu.make_async_copy(b_hbm.at[call_.at[pl.ds(ci * CG, CG)]],
                              bbuf, sb).wait()

    def compute(ci, abuf, bbuf):
        def group_body(g, _):
            ebias = ci * CG + g * 16
            cvec = call_[pl.ds(ebias, 16)]
            evec = _bfr2(eall[pl.ds(ebias, 16)])
            sv = zero16
            for e in range(16):
                eidx = g * 16 + e
                eav = _vgather(evec, jnp.full((16,), e, jnp.int32))
                ts = []
                for k in range(8):
                    va = abuf[eidx, pl.ds(k * 16, 16)]
                    vb = bbuf[eidx, pl.ds(k * 16, 16)]
                    ts.append(jnp.maximum(va + vb + eav * w3[k], 0.0))
                ps = [_bfr2(ts[k]) * w2[k] for k in range(8)]
                acc = ((ps[0] + ps[1]) + (ps[2] + ps[3])) + (
                    (ps[4] + ps[5]) + (ps[6] + ps[7]))
                for d in (8, 4, 2, 1):
                    acc = acc + _vgather(acc, iota16 ^ d)
                sv = jnp.where(iota16 == e, acc, sv)
            sv = sv + be2v
            seall[pl.ds(ebias, 16)] = sv

            sums = sv
            maxs = sv
            cnts = jnp.ones((16,), jnp.float32)
            bad = jnp.zeros((16,), jnp.bool_)
            for d in range(1, 16):
                idx = (iota16 + d) & 15
                same = cvec == _vgather(cvec, idx)
                svg = _vgather(sv, idx)
                wrap = iota16 >= (16 - d)
                bad = bad | (same & wrap)
                sums = sums + jnp.where(same, svg, 0.0)
                cnts = cnts + jnp.where(same, 1.0, 0.0)
                maxs = jnp.where(same, jnp.maximum(maxs, svg), maxs)
            first = ~bad
            plsc.addupdate_scatter(sumbin, [cvec], sums, mask=first)
            plsc.addupdate_scatter(cntbin, [cvec], cnts, mask=first)
            old = plsc.load_gather(maxbin, [cvec])
            plsc.store_scatter(maxbin, [cvec], jnp.maximum(old, maxs),
                               mask=first)
            return 0

        lax.fori_loop(0, NGRP, group_body, 0)

    issue(0, abufa, bbufa, semaa, semab)

    def pair_body(g, _):
        c0 = 2 * g
        c1 = c0 + 1

        @pl.when(c1 < NCHUNK)
        def _():
            issue(c1, abufb, bbufb, semba, sembb)

        wait(c0, abufa, bbufa, semaa, semab)
        compute(c0, abufa, bbufa)

        @pl.when(c1 < NCHUNK)
        def _():
            @pl.when(c1 + 1 < NCHUNK)
            def _():
                issue(c1 + 1, abufa, bbufa, semaa, semab)

            wait(c1, abufb, bbufb, semba, sembb)
            compute(c1, abufb, bbufb)

        return 0

    lax.fori_loop(0, (NCHUNK + 1) // 2, pair_body, 0)

    pltpu.sync_copy(seall, se_hbm.at[pl.ds(ebase, EPW)])
    pltpu.sync_copy(sumbin, sum_hbm.at[wid])
    pltpu.sync_copy(maxbin, max_hbm.at[wid])
    pltpu.sync_copy(cntbin, cnt_hbm.at[wid])


_edge_call = pl.kernel(
    _edge_body,
    out_type=(
        jax.ShapeDtypeStruct((NW, N), jnp.float32),
        jax.ShapeDtypeStruct((NW, N), jnp.float32),
        jax.ShapeDtypeStruct((NW, N), jnp.float32),
        jax.ShapeDtypeStruct((E,), jnp.float32),
    ),
    mesh=plsc.VectorSubcoreMesh(core_axis_name="c", subcore_axis_name="s"),
    compiler_params=pltpu.CompilerParams(needs_layout_passes=False),
    scratch_types=[
        pltpu.VMEM((N,), jnp.float32),
        pltpu.VMEM((N,), jnp.float32),
        pltpu.VMEM((N,), jnp.float32),
        pltpu.VMEM((EPW,), jnp.int32),
        pltpu.VMEM((EPW,), jnp.int32),
        pltpu.VMEM((EPW,), jnp.float32),
        pltpu.VMEM((EPW,), jnp.float32),
        pltpu.VMEM((CG, H), jnp.float32),
        pltpu.VMEM((CG, H), jnp.float32),
        pltpu.VMEM((CG, H), jnp.float32),
        pltpu.VMEM((CG, H), jnp.float32),
        pltpu.VMEM((3, H), jnp.float32),
        pltpu.SemaphoreType.DMA,
        pltpu.SemaphoreType.DMA,
        pltpu.SemaphoreType.DMA,
        pltpu.SemaphoreType.DMA,
    ],
)


BN = 2000
NB = N // BN
_HI = lax.Precision.HIGHEST


def _dot(a, b):
    return jnp.dot(a, b, preferred_element_type=jnp.float32, precision=_HI)


def _dot0(a, b):
    return lax.dot_general(a, b, (((0,), (0,)), ((), ())),
                           preferred_element_type=jnp.float32, precision=_HI)


def _full_spec(shape):
    return pl.BlockSpec(shape, lambda i: tuple(0 for _ in shape))


def _prep_body(x_ref, wa_ref, wb_ref, ba_ref, a_ref, b_ref):
    xv = _bfr(x_ref[...])
    a_ref[...] = _dot(xv, wa_ref[...]) + ba_ref[...]
    b_ref[...] = _dot(xv, wb_ref[...])


def _prep(x, wa, wb, ba):
    row_spec = pl.BlockSpec((BN, D), lambda i: (i, 0))
    return pl.pallas_call(
        _prep_body,
        grid=(NB,),
        in_specs=[row_spec, _full_spec((D, D)), _full_spec((D, D)),
                  _full_spec((1, H))],
        out_specs=(row_spec, row_spec),
        out_shape=(jax.ShapeDtypeStruct((N, D), jnp.float32),
                   jax.ShapeDtypeStruct((N, D), jnp.float32)),
    )(x, wa, wb, ba)


def _node_common(x_ref, ps_ref, pm_ref, pc_ref, batch_ref, u_ref,
                 wn1a_ref, wrow_ref, wu_ref, bn1_ref, wn2_ref, bn2_ref):
    s = jnp.sum(ps_ref[0], axis=0, keepdims=True)
    c = jnp.sum(pc_ref[0], axis=0, keepdims=True)
    m = jnp.max(pm_ref[0], axis=0, keepdims=True)
    m = jnp.where(c > 0.0, m, 0.0)
    mean = s / jnp.maximum(c, 1.0)

    wrow = wrow_ref[...]
    onehot = (batch_ref[...] ==
              lax.broadcasted_iota(jnp.int32, (BN, G), 1)).astype(jnp.float32)
    uw = _dot(_bfr(u_ref[...]), wu_ref[...])

    h = (_dot(_bfr(x_ref[...]), wn1a_ref[...])
         + _dot0(_bfr(s), wrow[0:1])
         + _dot0(_bfr(m), wrow[1:2])
         + _dot0(_bfr(mean), wrow[2:3])
         + _dot(onehot, uw)
         + bn1_ref[...])
    h = _bfr(jnp.maximum(h, 0.0))
    return _dot(h, wn2_ref[...]) + bn2_ref[...]


def _node_fused_body(x_ref, ps_ref, pm_ref, pc_ref, batch_ref, u_ref,
                     wn1a_ref, wrow_ref, wu_ref, bn1_ref, wn2_ref, bn2_ref,
                     wea_ref, web_ref, bea_ref,
                     xn_ref, a2_ref, b2_ref):
    xn = _node_common(x_ref, ps_ref, pm_ref, pc_ref, batch_ref, u_ref,
                      wn1a_ref, wrow_ref, wu_ref, bn1_ref, wn2_ref, bn2_ref)
    xn_ref[...] = xn
    xnr = _bfr(xn)
    a2_ref[...] = _dot(xnr, wea_ref[...]) + bea_ref[...]
    b2_ref[...] = _dot(xnr, web_ref[...])


def _node_plain_body(x_ref, ps_ref, pm_ref, pc_ref, batch_ref, u_ref,
                     wn1a_ref, wrow_ref, wu_ref, bn1_ref, wn2_ref, bn2_ref,
                     xn_ref):
    xn_ref[...] = _node_common(x_ref, ps_ref, pm_ref, pc_ref, batch_ref,
                               u_ref, wn1a_ref, wrow_ref, wu_ref, bn1_ref,
                               wn2_ref, bn2_ref)


def _node_specs():
    row_spec = pl.BlockSpec((BN, D), lambda i: (i, 0))
    part_spec = pl.BlockSpec((1, NW, BN), lambda i: (i, 0, 0))
    batch_spec = pl.BlockSpec((BN, 1), lambda i: (i, 0))
    wspecs = [_full_spec((G, 1)), _full_spec((D, D)), _full_spec((3, H)),
              _full_spec((1, H)), _full_spec((1, H)), _full_spec((D, D)),
              _full_spec((1, D))]
    return [row_spec, part_spec, part_spec, part_spec, batch_spec] + wspecs


def _node_fused(x, ps, pm, pc, batch2, u, wn1a, wrow, wu, bn1, wn2, bn2,
                wea, web, bea):
    row_spec = pl.BlockSpec((BN, D), lambda i: (i, 0))
    return pl.pallas_call(
        _node_fused_body,
        grid=(NB,),
        in_specs=_node_specs() + [_full_spec((D, D)), _full_spec((D, D)),
                                  _full_spec((1, H))],
        out_specs=(row_spec, row_spec, row_spec),
        out_shape=(jax.ShapeDtypeStruct((N, D), jnp.float32),
                   jax.ShapeDtypeStruct((N, D), jnp.float32),
                   jax.ShapeDtypeStruct((N, D), jnp.float32)),
    )(x, ps, pm, pc, batch2, u, wn1a, wrow, wu, bn1, wn2, bn2, wea, web, bea)


def _node_plain(x, ps, pm, pc, batch2, u, wn1a, wrow, wu, bn1, wn2, bn2):
    row_spec = pl.BlockSpec((BN, D), lambda i: (i, 0))
    return pl.pallas_call(
        _node_plain_body,
        grid=(NB,),
        in_specs=_node_specs(),
        out_specs=row_spec,
        out_shape=jax.ShapeDtypeStruct((N, D), jnp.float32),
    )(x, ps, pm, pc, batch2, u, wn1a, wrow, wu, bn1, wn2, bn2)


def _pool_body(x_ref, batch_ref, addp_ref, cnt_ref, maxp_ref):
    i = pl.program_id(0)
    xv = x_ref[...]
    bv = batch_ref[...]
    onehot = (bv == lax.broadcasted_iota(jnp.int32, (BN, G), 1)
              ).astype(jnp.float32)
    addp = _dot0(onehot, xv)
    cnt = _dot0(onehot, jnp.ones((BN, 1), jnp.float32))
    rows = []
    for g in range(G):
        rows.append(jnp.max(jnp.where(bv == g, xv, _F32MIN), axis=0,
                            keepdims=True))
    maxp = jnp.concatenate(rows, axis=0)

    @pl.when(i == 0)
    def _():
        addp_ref[...] = jnp.zeros((G, D), jnp.float32)
        cnt_ref[...] = jnp.zeros((G, 1), jnp.float32)
        maxp_ref[...] = jnp.full((G, D), _F32MIN, jnp.float32)

    addp_ref[...] += addp
    cnt_ref[...] += cnt
    maxp_ref[...] = jnp.maximum(maxp_ref[...], maxp)


def _pool(x, batch2):
    return pl.pallas_call(
        _pool_body,
        grid=(NB,),
        in_specs=[pl.BlockSpec((BN, D), lambda i: (i, 0)),
                  pl.BlockSpec((BN, 1), lambda i: (i, 0))],
        out_specs=(_full_spec((G, D)), _full_spec((G, 1)),
                   _full_spec((G, D))),
        out_shape=(jax.ShapeDtypeStruct((G, D), jnp.float32),
                   jax.ShapeDtypeStruct((G, 1), jnp.float32),
                   jax.ShapeDtypeStruct((G, D), jnp.float32)),
    )(x, batch2)


def _outmlp_body(addp_ref, cnt_ref, maxp_ref, u_ref,
                 w1a_ref, w1b_ref, w1c_ref, w1u_ref, b1_ref,
                 w2_ref, b2_ref, w3_ref, b3_ref, w4_ref, b4_ref, out_ref):
    addp = addp_ref[...]
    cnt = cnt_ref[...]
    maxp = jnp.where(cnt > 0.0, maxp_ref[...], 0.0)
    meanp = addp / jnp.maximum(cnt, 1.0)
    h = (_dot(_bfr(addp), w1a_ref[...]) + _dot(_bfr(meanp), w1b_ref[...])
         + _dot(_bfr(maxp), w1c_ref[...]) + _dot(_bfr(u_ref[...]),
                                                 w1u_ref[...])
         + b1_ref[...])
    h = _bfr(jnp.maximum(h, 0.0))
    h = _bfr(jnp.maximum(_dot(h, w2_ref[...]) + b2_ref[...], 0.0))
    h = _bfr(jnp.maximum(_dot(h, w3_ref[...]) + b3_ref[...], 0.0))
    out_ref[...] = _dot(h, w4_ref[...]) + b4_ref[...]


def _outmlp(addp, cnt, maxp, u, outw):
    return pl.pallas_call(
        _outmlp_body,
        out_shape=jax.ShapeDtypeStruct((G, 8), jnp.float32),
    )(addp, cnt, maxp, u, *outw)


def kernel(x, edge_attr, u, params, edge_index, batch):
    row = edge_index[0].astype(jnp.int32)
    col = edge_index[1].astype(jnp.int32)
    ea = edge_attr[:, 0].astype(jnp.float32)
    batch2 = batch.astype(jnp.int32).reshape(N, 1)

    L = params["layers"]

    def layer_w(p):
        wa = _bfr(p["We1"][:D])
        wb = _bfr(p["We1"][D:2 * D])
        wc = jnp.stack([_bfr(p["We1"][2 * D]), _bfr(p["We2"][:, 0]),
                        jnp.full((H,), p["be2"][0], jnp.float32)], axis=0)
        wn1a = _bfr(p["Wn1"][:D])
        wrow = _bfr(p["Wn1"][D:D + 3])
        wu = _bfr(p["Wn1"][D + 3:D + 4])
        bn1 = p["bn1"].reshape(1, H)
        bn2 = p["bn2"].reshape(1, D)
        ba = p["be1"].reshape(1, H)
        return wa, wb, wc, wn1a, wrow, wu, bn1, _bfr(p["Wn2"]), bn2, ba

    (wa0, wb0, wc0, wn1a0, wrow0, wu0, bn10, wn20, bn20, ba0) = layer_w(L[0])
    (wa1, wb1, wc1, wn1a1, wrow1, wu1, bn11, wn21, bn21, ba1) = layer_w(L[1])

    (W1, b1), (W2, b2), (W3, b3), (W4, b4) = params["out"]
    W1, W2, W3, W4 = _bfr(W1), _bfr(W2), _bfr(W3), _bfr(W4)
    w4p = jnp.zeros((H, 8), jnp.float32).at[:, :2].set(W4)
    b4p = jnp.zeros((1, 8), jnp.float32).at[0, :2].set(b4)
    outw = (W1[:D], W1[D:2 * D], W1[2 * D:3 * D], W1[3 * D:3 * D + 1],
            b1.reshape(1, H), W2, b2.reshape(1, H), W3, b3.reshape(1, H),
            w4p, b4p)

    def part3(p):
        return jnp.transpose(p.reshape(NW, NB, BN), (1, 0, 2))

    A, B = _prep(x, wa0, wb0, ba0)
    ps, pm, pc, se = _edge_call(A, B, row, col, ea, wc0)
    x1, A2, B2 = _node_fused(x, part3(ps), part3(pm), part3(pc), batch2, u,
                             wn1a0, wrow0, wu0, bn10, wn20, bn20,
                             wa1, wb1, ba1)
    ps, pm, pc, _ = _edge_call(A2, B2, row, col, se, wc1)
    x2 = _node_plain(x1, part3(ps), part3(pm), part3(pc), batch2, u,
                     wn1a1, wrow1, wu1, bn11, wn21, bn21)
    addp, cnt, maxp = _pool(x2, batch2)
    out = _outmlp(addp, cnt, maxp, u, outw)
    return out[:, :2]
